# baseline (speedup 1.0000x reference)
import functools

import jax
import jax.numpy as jnp
from jax.experimental import pallas as pl
from jax.experimental.pallas import tpu as pltpu

_N = 20000
_C = 128
_R = 160
_NPAD = _R * _C
_IMTOP = 200
_IOU_THR = 0.45
_SCORE_THR = 0.01
_NEG_INF = float("-inf")


def _comb(a, b):
    take_b = (b[0] > a[0]) | ((b[0] == a[0]) & (b[1] < a[1]))
    return tuple(jnp.where(take_b, xb, xa) for xa, xb in zip(a, b))


def _argmax_tuple(t):
    parts = [tuple(arr[k * 8:(k + 1) * 8, :] for arr in t)
             for k in range(_R // 8)]
    while len(parts) > 1:
        nxt = [_comb(parts[p], parts[p + 1])
               for p in range(0, len(parts) - 1, 2)]
        if len(parts) % 2:
            nxt.append(parts[-1])
        parts = nxt
    cur = parts[0]
    for sh in (4, 2, 1):
        rolled = tuple(jnp.concatenate([arr[sh:, :], arr[:sh, :]], axis=0)
                       for arr in cur)
        cur = _comb(cur, rolled)
    cur = tuple(arr[0:1, :] for arr in cur)
    for sh in (64, 32, 16, 8, 4, 2, 1):
        rolled = tuple(jnp.concatenate([arr[:, sh:], arr[:, :sh]], axis=1)
                       for arr in cur)
        cur = _comb(cur, rolled)
    return cur


def _nms_kernel(bxs_ref, sc_ref, out_ref, s_ref, a2_ref):
    s_raw = sc_ref[...]
    sv0 = jnp.where(s_raw > _SCORE_THR, s_raw, _NEG_INF)
    s_ref[...] = sv0

    row_i = jax.lax.broadcasted_iota(jnp.int32, (_R, _C), 0)
    col_i = jax.lax.broadcasted_iota(jnp.int32, (_R, _C), 1)
    idx_f = (row_i * _C + col_i).astype(jnp.float32)
    lane = jax.lax.broadcasted_iota(jnp.int32, (1, 128), 1)

    x1 = bxs_ref[0, :, :]
    y1 = bxs_ref[1, :, :]
    x2 = bxs_ref[2, :, :]
    y2 = bxs_ref[3, :, :]
    a2_ref[...] = (x2 - x1) * (y2 - y1)

    sel_init = _argmax_tuple((sv0, idx_f, x1, y1, x2, y2))

    zc = jnp.zeros((1, _C), jnp.float32)
    ninf_c = jnp.full((1, _C), _NEG_INF, jnp.float32)

    def body(t, carry):
        (m, j, bx1, by1, bx2, by2, j0, s0, b0x1, b0y1, b0x2, b0y2) = carry
        empty = m == _NEG_INF
        jj = jnp.where(empty, j0, j)
        ex1 = jnp.where(empty, b0x1, bx1)
        ey1 = jnp.where(empty, b0y1, by1)
        ex2 = jnp.where(empty, b0x2, bx2)
        ey2 = jnp.where(empty, b0y2, by2)

        x1 = bxs_ref[0, :, :]
        y1 = bxs_ref[1, :, :]
        x2 = bxs_ref[2, :, :]
        y2 = bxs_ref[3, :, :]

        xx1 = jnp.maximum(ex1, x1)
        yy1 = jnp.maximum(ey1, y1)
        xx2 = jnp.minimum(ex2, x2)
        yy2 = jnp.minimum(ey2, y2)
        inter = jnp.maximum(xx2 - xx1, 0.0) * jnp.maximum(yy2 - yy1, 0.0)
        a1 = (ex2 - ex1) * (ey2 - ey1)
        iou = inter / (a1 + a2_ref[...] - inter + 1e-9)

        sv = s_ref[...]
        s_new = jnp.where((iou > _IOU_THR) | (idx_f == jj), _NEG_INF, sv)
        s_ref[...] = s_new

        nm, nj, nbx1, nby1, nbx2, nby2 = _argmax_tuple(
            (s_new, idx_f, x1, y1, x2, y2))

        sel_score = jnp.where(empty, s0, m)
        row = jnp.zeros((1, 128), jnp.float32)
        row = jnp.where(lane == 0, ex1, row)
        row = jnp.where(lane == 1, ey1, row)
        row = jnp.where(lane == 2, ex2, row)
        row = jnp.where(lane == 3, ey2, row)
        row = jnp.where(lane == 4, sel_score, row)
        out_ref[pl.ds(t, 1), :] = row

        first = t == 0
        j0 = jnp.where(first, jj, j0)
        s0 = jnp.where(first, sel_score, s0)
        b0x1 = jnp.where(first, ex1, b0x1)
        b0y1 = jnp.where(first, ey1, b0y1)
        b0x2 = jnp.where(first, ex2, b0x2)
        b0y2 = jnp.where(first, ey2, b0y2)
        return (nm, nj, nbx1, nby1, nbx2, nby2, j0, s0,
                b0x1, b0y1, b0x2, b0y2)

    jax.lax.fori_loop(
        0, _IMTOP, body,
        (*sel_init, zc, ninf_c, zc, zc, zc, zc))


@functools.partial(jax.jit, static_argnames=())
def _run(boxes, scores):
    bxs = jnp.pad(boxes.T, ((0, 0), (0, _NPAD - _N))).reshape(4, _R, _C)
    sc = jnp.pad(scores, (0, _NPAD - _N)).reshape(_R, _C)
    out = pl.pallas_call(
        _nms_kernel,
        out_shape=jax.ShapeDtypeStruct((_IMTOP, 128), jnp.float32),
        scratch_shapes=[pltpu.VMEM((_R, _C), jnp.float32),
                        pltpu.VMEM((_R, _C), jnp.float32)],
    )(bxs, sc)
    return out[:, :5]


def kernel(boxes, scores, imtop):
    del imtop
    return _run(boxes, scores)
